# baseline (speedup 1.0000x reference)
import functools

import numpy as np
import jax
import jax.numpy as jnp
from jax import lax
from jax.experimental import pallas as pl
from jax.experimental.pallas import tpu as pltpu
from jax.experimental.pallas import tpu_sc as plsc

NUM_HE = 5000
RPAD = 5120
CHUNK = 256
DW = 128
NCORES = 2
NSUB = 16
NW = NCORES * NSUB
CPW = 42
NDUO = CPW // 2
EPAD = NW * CPW * CHUNK
ROWS_PER_TILE = RPAD // NSUB
_BNS = float(1.0 / np.sqrt(1.0 + 1e-5))
_HIGH = lax.Precision.HIGHEST


def _mesh():
    return plsc.VectorSubcoreMesh(core_axis_name="c", subcore_axis_name="s")


_SC_PARAMS = pltpu.CompilerParams(use_tc_tiling_on_sc=False)


def _fill(buf, width, value):
    def fr(i, carry):
        for j in range(width // 16):
            buf[i, pl.ds(j * 16, 16)] = jnp.full((16,), value, jnp.float32)
        return carry

    lax.fori_loop(0, CHUNK, fr, 0)


def _zero_acc(buf, acc, base_r):
    done = 0
    while done < ROWS_PER_TILE:
        n = min(CHUNK, ROWS_PER_TILE - done)
        pltpu.sync_copy(buf.at[pl.ds(0, n)], acc.at[pl.ds(base_r + done, n)])
        done += n


@functools.lru_cache(maxsize=None)
def _sc_pass(width):
    @functools.partial(
        pl.kernel,
        mesh=_mesh(),
        out_type=jax.ShapeDtypeStruct((NCORES, RPAD, width), jnp.float32),
        scratch_types=[
            pltpu.VMEM((CPW, CHUNK), jnp.int32),
            pltpu.VMEM((CPW, CHUNK), jnp.int32),
            pltpu.VMEM((CHUNK, width), jnp.float32),
            pltpu.VMEM((CHUNK, width), jnp.float32),
            pltpu.VMEM_SHARED((RPAD, width), jnp.float32),
            pltpu.SemaphoreType.DMA,
            pltpu.SemaphoreType.DMA,
            pltpu.SemaphoreType.DMA,
            pltpu.SemaphoreType.DMA,
        ],
        compiler_params=_SC_PARAMS,
    )
    def k(table_hbm, src_hbm, dst_hbm, out_hbm, src_all, dst_all,
          b0, b1, acc_sh, g0, g1, s0, s1):
        cid = lax.axis_index("c")
        sid = lax.axis_index("s")
        wid = sid * NCORES + cid
        base_r = sid * ROWS_PER_TILE
        bufs = (b0, b1)
        gsem = (g0, g1)
        ssem = (s0, s1)

        pltpu.sync_copy(src_hbm.at[wid], src_all)
        pltpu.sync_copy(dst_hbm.at[wid], dst_all)
        _fill(b0, width, 0.0)
        _zero_acc(b0, acc_sh, base_r)
        plsc.subcore_barrier()

        def fire_g(c, b):
            pltpu.async_copy(table_hbm.at[src_all.at[c]], bufs[b], gsem[b])

        def wait_g(c, b):
            pltpu.make_async_copy(table_hbm.at[src_all.at[c]], bufs[b], gsem[b]).wait()

        def fire_s(c, b):
            pltpu.async_copy(bufs[b], acc_sh.at[dst_all.at[c]], ssem[b], add=True)

        def wait_s(c, b):
            pltpu.make_async_copy(bufs[b], acc_sh.at[dst_all.at[c]], ssem[b]).wait()

        fire_g(0, 0)
        fire_g(1, 1)

        def duo(t, carry):
            c0 = 2 * t
            wait_g(c0, 0)
            fire_s(c0, 0)
            wait_g(c0 + 1, 1)
            fire_s(c0 + 1, 1)
            wait_s(c0, 0)

            @pl.when(c0 + 2 < CPW)
            def _():
                fire_g(c0 + 2, 0)

            wait_s(c0 + 1, 1)

            @pl.when(c0 + 3 < CPW)
            def _():
                fire_g(c0 + 3, 1)

            return carry

        lax.fori_loop(0, NDUO, duo, 0)
        plsc.subcore_barrier()
        pltpu.sync_copy(acc_sh.at[pl.ds(base_r, ROWS_PER_TILE)],
                        out_hbm.at[cid, pl.ds(base_r, ROWS_PER_TILE)])

    return k


def _dot(a, b):
    return lax.dot_general(a, b, (((1,), (0,)), ((), ())),
                           precision=_HIGH, preferred_element_type=jnp.float32)


def _tc_matmul(a, w, rows, br):
    K, N = w.shape

    def body(a_ref, w_ref, o_ref):
        o_ref[...] = _dot(a_ref[...], w_ref[...])

    return pl.pallas_call(
        body, grid=(rows // br,),
        in_specs=[pl.BlockSpec((br, K), lambda i: (i, 0)),
                  pl.BlockSpec((K, N), lambda i: (0, 0))],
        out_specs=pl.BlockSpec((br, N), lambda i: (i, 0)),
        out_shape=jax.ShapeDtypeStruct((rows, N), jnp.float32),
    )(a, w)


def _inv_col(d_ref):
    d = d_ref[0][:, 0:1] + d_ref[1][:, 0:1]
    return jnp.where(d > 0, 1.0 / d, 0.0)


def _tc_mid(m_parts, b_parts, C):
    BR = 1024

    def body(m_ref, b_ref, o_ref):
        o_ref[...] = (m_ref[0] + m_ref[1]) * _inv_col(b_ref)

    return pl.pallas_call(
        body, grid=(RPAD // BR,),
        in_specs=[pl.BlockSpec((NCORES, BR, C), lambda i: (0, i, 0)),
                  pl.BlockSpec((NCORES, BR, DW), lambda i: (0, i, 0))],
        out_specs=pl.BlockSpec((BR, C), lambda i: (i, 0)),
        out_shape=jax.ShapeDtypeStruct((RPAD, C), jnp.float32),
    )(m_parts, b_parts)


def _conv_out(p_ref, d_ref, bias):
    s = p_ref[0] + p_ref[1]
    return jax.nn.relu(s * _inv_col(d_ref) + bias)


def _tc_stage1(parts, d_parts, x0, bc1, w1, b1, w2, b2, g, beta):
    C = 256
    BR = 1000

    def body(p_ref, d_ref, x0_ref, bc1_r, w1_r, b1_r, w2_r, b2_r, g_r, beta_r, o_ref):
        x1 = _conv_out(p_ref, d_ref, bc1_r[...])
        t = jax.nn.relu(_dot(x1, w1_r[...]) + b1_r[...])
        s1 = jax.nn.sigmoid(_dot(t, w2_r[...]) + b2_r[...])
        o_ref[...] = (x1 + s1 + x0_ref[...]) * (g_r[...] * _BNS) + beta_r[...]

    return pl.pallas_call(
        body, grid=(5000 // BR,),
        in_specs=[pl.BlockSpec((NCORES, BR, C), lambda i: (0, i, 0)),
                  pl.BlockSpec((NCORES, BR, DW), lambda i: (0, i, 0)),
                  pl.BlockSpec((BR, C), lambda i: (i, 0)),
                  pl.BlockSpec((1, C), lambda i: (0, 0)),
                  pl.BlockSpec((C, 128), lambda i: (0, 0)),
                  pl.BlockSpec((1, 128), lambda i: (0, 0)),
                  pl.BlockSpec((128, C), lambda i: (0, 0)),
                  pl.BlockSpec((1, C), lambda i: (0, 0)),
                  pl.BlockSpec((1, C), lambda i: (0, 0)),
                  pl.BlockSpec((1, C), lambda i: (0, 0))],
        out_specs=pl.BlockSpec((BR, C), lambda i: (i, 0)),
        out_shape=jax.ShapeDtypeStruct((5000, C), jnp.float32),
    )(parts, d_parts, x0, bc1, w1, b1, w2, b2, g, beta)


def _tc_stageL(parts, d_parts, x0, bconv, g, beta, w1, b1, w2, b2):
    C = 256
    BR = 1000

    def body(p_ref, d_ref, x0_ref, bc_r, g_r, beta_r, w1_r, b1_r, w2_r, b2_r, o_ref):
        c = _conv_out(p_ref, d_ref, bc_r[...])
        t = (0.8 * c + 0.2 * x0_ref[...]) * (g_r[...] * _BNS) + beta_r[...]
        s = jax.nn.relu(_dot(t, w1_r[...]) + b1_r[...])
        o_ref[...] = jax.nn.sigmoid(_dot(s, w2_r[...]) + b2_r[...])

    return pl.pallas_call(
        body, grid=(5000 // BR,),
        in_specs=[pl.BlockSpec((NCORES, BR, C), lambda i: (0, i, 0)),
                  pl.BlockSpec((NCORES, BR, DW), lambda i: (0, i, 0)),
                  pl.BlockSpec((BR, C), lambda i: (i, 0)),
                  pl.BlockSpec((1, C), lambda i: (0, 0)),
                  pl.BlockSpec((1, C), lambda i: (0, 0)),
                  pl.BlockSpec((1, C), lambda i: (0, 0)),
                  pl.BlockSpec((C, 128), lambda i: (0, 0)),
                  pl.BlockSpec((1, 128), lambda i: (0, 0)),
                  pl.BlockSpec((128, C), lambda i: (0, 0)),
                  pl.BlockSpec((1, C), lambda i: (0, 0))],
        out_specs=pl.BlockSpec((BR, C), lambda i: (i, 0)),
        out_shape=jax.ShapeDtypeStruct((5000, C), jnp.float32),
    )(parts, d_parts, x0, bconv, g, beta, w1, b1, w2, b2)


def _tc_final(parts, d_parts, bc3):
    C = 128
    BR = 1000

    def body(p_ref, d_ref, b_r, o_ref):
        i = pl.program_id(0)
        lo = _conv_out(p_ref, d_ref, b_r[...])
        hi = jnp.broadcast_to(jax.nn.relu(b_r[...]), (BR, C))
        o_ref[...] = jnp.where(i < 5, lo, hi)

    return pl.pallas_call(
        body, grid=(10000 // BR,),
        in_specs=[pl.BlockSpec((NCORES, BR, C), lambda i: (0, jnp.minimum(i, 4), 0)),
                  pl.BlockSpec((NCORES, BR, DW), lambda i: (0, jnp.minimum(i, 4), 0)),
                  pl.BlockSpec((1, C), lambda i: (0, 0))],
        out_specs=pl.BlockSpec((BR, C), lambda i: (i, 0)),
        out_shape=jax.ShapeDtypeStruct((10000, C), jnp.float32),
    )(parts, d_parts, bc3)


def kernel(x, edge_index, params):
    mask = jax.random.bernoulli(jax.random.key(42), 0.8, edge_index.shape)
    edge = edge_index * mask.astype(edge_index.dtype)
    npad = EPAD - edge_index.shape[1]
    pad = jnp.full((npad,), NUM_HE, jnp.int32)
    node = jnp.concatenate([edge[0], pad]).reshape(NW, CPW, CHUNK)
    he = jnp.concatenate([edge[1], pad]).reshape(NW, CPW, CHUNK)

    ones_tab = jnp.ones((RPAD, DW), jnp.float32)
    d_parts = _sc_pass(DW)(ones_tab, node, node)
    b_parts = _sc_pass(DW)(ones_tab, he, he)

    x_lo = x[:NUM_HE]
    x0 = _tc_matmul(x_lo, params['W_ln'], 5000, 1000)

    def row(v):
        return v[None, :]

    def hconv(h5000, wmat):
        C = wmat.shape[1]
        h_pad = jnp.pad(h5000, ((0, RPAD - NUM_HE), (0, 0)))
        tab = _tc_matmul(h_pad, wmat, RPAD, 1024)
        m_parts = jnp.concatenate(
            [_sc_pass(128)(tab[:, c0:c0 + 128], node, he)
             for c0 in range(0, C, 128)], axis=2)
        m2 = _tc_mid(m_parts, b_parts, C)
        return jnp.concatenate(
            [_sc_pass(128)(m2[:, c0:c0 + 128], he, node)
             for c0 in range(0, C, 128)], axis=2)

    oB = hconv(x_lo, params['W_c1'])
    h = _tc_stage1(oB, d_parts, x0, row(params['b_c1']),
                   params['W_fc1'], row(params['b_fc1']),
                   params['W_fc2'], row(params['b_fc2']),
                   row(params['g1']), row(params['beta1']))
    for i in range(3):
        oB = hconv(h, params['W_convs'][i])
        h = _tc_stageL(oB, d_parts, x0, row(params['b_convs'][i]),
                       row(params['gs'][i]), row(params['betas'][i]),
                       params['W_fc1s'][i], row(params['b_fc1s'][i]),
                       params['W_fc2s'][i], row(params['b_fc2s'][i]))
    oB = hconv(h, params['W_c3'])
    return _tc_final(oB, d_parts, row(params['b_c3']))


# speedup vs baseline: 1.9404x; 1.9404x over previous
import functools

import numpy as np
import jax
import jax.numpy as jnp
from jax import lax
from jax.experimental import pallas as pl
from jax.experimental.pallas import tpu as pltpu
from jax.experimental.pallas import tpu_sc as plsc

NUM_HE = 5000
RPAD = 5120
CHUNK = 64
DW = 16
NCORES = 2
NSUB = 16
NW = NCORES * NSUB
CPW = 168
NDUO = CPW // 2
EPAD = NW * CPW * CHUNK
ROWS_PER_TILE = RPAD // NSUB
_BNS = float(1.0 / np.sqrt(1.0 + 1e-5))
_HIGH = lax.Precision.HIGHEST


def _mesh():
    return plsc.VectorSubcoreMesh(core_axis_name="c", subcore_axis_name="s")


_SC_PARAMS = pltpu.CompilerParams(use_tc_tiling_on_sc=False)


def _fill(buf, width, value):
    def fr(i, carry):
        for j in range(width // 16):
            buf[i, pl.ds(j * 16, 16)] = jnp.full((16,), value, jnp.float32)
        return carry

    lax.fori_loop(0, CHUNK, fr, 0)


def _zero_acc(buf, acc, base_r):
    done = 0
    while done < ROWS_PER_TILE:
        n = min(CHUNK, ROWS_PER_TILE - done)
        pltpu.sync_copy(buf.at[pl.ds(0, n)], acc.at[pl.ds(base_r + done, n)])
        done += n


@functools.partial(
    pl.kernel,
    mesh=_mesh(),
    out_type=(jax.ShapeDtypeStruct((NCORES, RPAD, DW), jnp.float32),
              jax.ShapeDtypeStruct((NCORES, RPAD, DW), jnp.float32)),
    scratch_types=[
        pltpu.VMEM((CPW, CHUNK), jnp.int32),
        pltpu.VMEM((CPW, CHUNK), jnp.int32),
        pltpu.VMEM((CHUNK, DW), jnp.float32),
        pltpu.VMEM_SHARED((RPAD, DW), jnp.float32),
        pltpu.VMEM_SHARED((RPAD, DW), jnp.float32),
        pltpu.SemaphoreType.DMA,
        pltpu.SemaphoreType.DMA,
    ],
    compiler_params=_SC_PARAMS,
)
def _sc_degrees(src_hbm, dst_hbm, outD, outB, src_all, dst_all, buf_v, accD, accB, semD, semB):
    cid = lax.axis_index("c")
    sid = lax.axis_index("s")
    wid = sid * NCORES + cid
    base_r = sid * ROWS_PER_TILE

    pltpu.sync_copy(src_hbm.at[wid], src_all)
    pltpu.sync_copy(dst_hbm.at[wid], dst_all)
    _fill(buf_v, DW, 0.0)
    _zero_acc(buf_v, accD, base_r)
    _zero_acc(buf_v, accB, base_r)
    _fill(buf_v, DW, 1.0)
    plsc.subcore_barrier()

    def body(c, carry):
        pltpu.async_copy(buf_v, accD.at[src_all.at[c]], semD, add=True)
        pltpu.async_copy(buf_v, accB.at[dst_all.at[c]], semB, add=True)

        @pl.when(c > 0)
        def _():
            pltpu.make_async_copy(buf_v, accD.at[src_all.at[c - 1]], semD).wait()
            pltpu.make_async_copy(buf_v, accB.at[dst_all.at[c - 1]], semB).wait()

        return carry

    lax.fori_loop(0, CPW, body, 0)
    pltpu.make_async_copy(buf_v, accD.at[src_all.at[CPW - 1]], semD).wait()
    pltpu.make_async_copy(buf_v, accB.at[dst_all.at[CPW - 1]], semB).wait()
    plsc.subcore_barrier()
    pltpu.sync_copy(accD.at[pl.ds(base_r, ROWS_PER_TILE)],
                    outD.at[cid, pl.ds(base_r, ROWS_PER_TILE)])
    pltpu.sync_copy(accB.at[pl.ds(base_r, ROWS_PER_TILE)],
                    outB.at[cid, pl.ds(base_r, ROWS_PER_TILE)])


@functools.lru_cache(maxsize=None)
def _sc_pass(width):
    @functools.partial(
        pl.kernel,
        mesh=_mesh(),
        out_type=jax.ShapeDtypeStruct((NCORES, RPAD, width), jnp.float32),
        scratch_types=[
            pltpu.VMEM((CHUNK,), jnp.int32),
            pltpu.VMEM((CHUNK,), jnp.int32),
            pltpu.VMEM((CHUNK,), jnp.int32),
            pltpu.VMEM((CHUNK,), jnp.int32),
            pltpu.VMEM((CHUNK, width), jnp.float32),
            pltpu.VMEM((CHUNK, width), jnp.float32),
            pltpu.VMEM_SHARED((RPAD, width), jnp.float32),
            pltpu.SemaphoreType.DMA,
            pltpu.SemaphoreType.DMA,
            pltpu.SemaphoreType.DMA,
            pltpu.SemaphoreType.DMA,
        ],
        compiler_params=_SC_PARAMS,
    )
    def k(table_hbm, src_hbm, dst_hbm, out_hbm, sv0, sv1, dv0, dv1,
          b0, b1, acc_sh, g0, g1, s0, s1):
        cid = lax.axis_index("c")
        sid = lax.axis_index("s")
        wid = sid * NCORES + cid
        base_r = sid * ROWS_PER_TILE
        bufs = (b0, b1)
        srcv = (sv0, sv1)
        dstv = (dv0, dv1)
        gsem = (g0, g1)
        ssem = (s0, s1)

        _fill(b0, width, 0.0)
        _zero_acc(b0, acc_sh, base_r)
        plsc.subcore_barrier()

        def load_idx(c, b):
            pltpu.sync_copy(src_hbm.at[wid, c], srcv[b])
            pltpu.sync_copy(dst_hbm.at[wid, c], dstv[b])

        def fire_g(c, b):
            pltpu.async_copy(table_hbm.at[srcv[b]], bufs[b], gsem[b])

        def wait_g(b):
            pltpu.make_async_copy(table_hbm.at[srcv[b]], bufs[b], gsem[b]).wait()

        def fire_s(c, b):
            pltpu.async_copy(bufs[b], acc_sh.at[dstv[b]], ssem[b], add=True)

        def wait_s(b):
            pltpu.make_async_copy(bufs[b], acc_sh.at[dstv[b]], ssem[b]).wait()

        load_idx(0, 0)
        fire_g(0, 0)
        load_idx(1, 1)
        fire_g(1, 1)

        def duo(t, carry):
            c0 = 2 * t
            wait_g(0)
            fire_s(c0, 0)
            wait_g(1)
            fire_s(c0 + 1, 1)
            wait_s(0)

            @pl.when(c0 + 2 < CPW)
            def _():
                load_idx(c0 + 2, 0)
                fire_g(c0 + 2, 0)

            wait_s(1)

            @pl.when(c0 + 3 < CPW)
            def _():
                load_idx(c0 + 3, 1)
                fire_g(c0 + 3, 1)

            return carry

        lax.fori_loop(0, NDUO, duo, 0)
        plsc.subcore_barrier()
        pltpu.sync_copy(acc_sh.at[pl.ds(base_r, ROWS_PER_TILE)],
                        out_hbm.at[cid, pl.ds(base_r, ROWS_PER_TILE)])

    return k


def _dot(a, b):
    return lax.dot_general(a, b, (((1,), (0,)), ((), ())),
                           precision=_HIGH, preferred_element_type=jnp.float32)


def _tc_matmul(a, w, rows, br):
    K, N = w.shape

    def body(a_ref, w_ref, o_ref):
        o_ref[...] = _dot(a_ref[...], w_ref[...])

    return pl.pallas_call(
        body, grid=(rows // br,),
        in_specs=[pl.BlockSpec((br, K), lambda i: (i, 0)),
                  pl.BlockSpec((K, N), lambda i: (0, 0))],
        out_specs=pl.BlockSpec((br, N), lambda i: (i, 0)),
        out_shape=jax.ShapeDtypeStruct((rows, N), jnp.float32),
    )(a, w)


def _inv_col(d_ref):
    d = d_ref[0][:, 0:1] + d_ref[1][:, 0:1]
    return jnp.where(d > 0, 1.0 / d, 0.0)


def _tc_mid(m_parts, b_parts, C):
    BR = 1024

    def body(m_ref, b_ref, o_ref):
        o_ref[...] = (m_ref[0] + m_ref[1]) * _inv_col(b_ref)

    return pl.pallas_call(
        body, grid=(RPAD // BR,),
        in_specs=[pl.BlockSpec((NCORES, BR, C), lambda i: (0, i, 0)),
                  pl.BlockSpec((NCORES, BR, DW), lambda i: (0, i, 0))],
        out_specs=pl.BlockSpec((BR, C), lambda i: (i, 0)),
        out_shape=jax.ShapeDtypeStruct((RPAD, C), jnp.float32),
    )(m_parts, b_parts)


def _conv_out(p_ref, d_ref, bias):
    s = p_ref[0] + p_ref[1]
    return jax.nn.relu(s * _inv_col(d_ref) + bias)


def _tc_stage1(parts, d_parts, x0, bc1, w1, b1, w2, b2, g, beta):
    C = 256
    BR = 1000

    def body(p_ref, d_ref, x0_ref, bc1_r, w1_r, b1_r, w2_r, b2_r, g_r, beta_r, o_ref):
        x1 = _conv_out(p_ref, d_ref, bc1_r[...])
        t = jax.nn.relu(_dot(x1, w1_r[...]) + b1_r[...])
        s1 = jax.nn.sigmoid(_dot(t, w2_r[...]) + b2_r[...])
        o_ref[...] = (x1 + s1 + x0_ref[...]) * (g_r[...] * _BNS) + beta_r[...]

    return pl.pallas_call(
        body, grid=(5000 // BR,),
        in_specs=[pl.BlockSpec((NCORES, BR, C), lambda i: (0, i, 0)),
                  pl.BlockSpec((NCORES, BR, DW), lambda i: (0, i, 0)),
                  pl.BlockSpec((BR, C), lambda i: (i, 0)),
                  pl.BlockSpec((1, C), lambda i: (0, 0)),
                  pl.BlockSpec((C, 128), lambda i: (0, 0)),
                  pl.BlockSpec((1, 128), lambda i: (0, 0)),
                  pl.BlockSpec((128, C), lambda i: (0, 0)),
                  pl.BlockSpec((1, C), lambda i: (0, 0)),
                  pl.BlockSpec((1, C), lambda i: (0, 0)),
                  pl.BlockSpec((1, C), lambda i: (0, 0))],
        out_specs=pl.BlockSpec((BR, C), lambda i: (i, 0)),
        out_shape=jax.ShapeDtypeStruct((5000, C), jnp.float32),
    )(parts, d_parts, x0, bc1, w1, b1, w2, b2, g, beta)


def _tc_stageL(parts, d_parts, x0, bconv, g, beta, w1, b1, w2, b2):
    C = 256
    BR = 1000

    def body(p_ref, d_ref, x0_ref, bc_r, g_r, beta_r, w1_r, b1_r, w2_r, b2_r, o_ref):
        c = _conv_out(p_ref, d_ref, bc_r[...])
        t = (0.8 * c + 0.2 * x0_ref[...]) * (g_r[...] * _BNS) + beta_r[...]
        s = jax.nn.relu(_dot(t, w1_r[...]) + b1_r[...])
        o_ref[...] = jax.nn.sigmoid(_dot(s, w2_r[...]) + b2_r[...])

    return pl.pallas_call(
        body, grid=(5000 // BR,),
        in_specs=[pl.BlockSpec((NCORES, BR, C), lambda i: (0, i, 0)),
                  pl.BlockSpec((NCORES, BR, DW), lambda i: (0, i, 0)),
                  pl.BlockSpec((BR, C), lambda i: (i, 0)),
                  pl.BlockSpec((1, C), lambda i: (0, 0)),
                  pl.BlockSpec((1, C), lambda i: (0, 0)),
                  pl.BlockSpec((1, C), lambda i: (0, 0)),
                  pl.BlockSpec((C, 128), lambda i: (0, 0)),
                  pl.BlockSpec((1, 128), lambda i: (0, 0)),
                  pl.BlockSpec((128, C), lambda i: (0, 0)),
                  pl.BlockSpec((1, C), lambda i: (0, 0))],
        out_specs=pl.BlockSpec((BR, C), lambda i: (i, 0)),
        out_shape=jax.ShapeDtypeStruct((5000, C), jnp.float32),
    )(parts, d_parts, x0, bconv, g, beta, w1, b1, w2, b2)


def _tc_final(parts, d_parts, bc3):
    C = 128
    BR = 1000

    def body(p_ref, d_ref, b_r, o_ref):
        i = pl.program_id(0)
        lo = _conv_out(p_ref, d_ref, b_r[...])
        hi = jnp.broadcast_to(jax.nn.relu(b_r[...]), (BR, C))
        o_ref[...] = jnp.where(i < 5, lo, hi)

    return pl.pallas_call(
        body, grid=(10000 // BR,),
        in_specs=[pl.BlockSpec((NCORES, BR, C), lambda i: (0, jnp.minimum(i, 4), 0)),
                  pl.BlockSpec((NCORES, BR, DW), lambda i: (0, jnp.minimum(i, 4), 0)),
                  pl.BlockSpec((1, C), lambda i: (0, 0))],
        out_specs=pl.BlockSpec((BR, C), lambda i: (i, 0)),
        out_shape=jax.ShapeDtypeStruct((10000, C), jnp.float32),
    )(parts, d_parts, bc3)


def kernel(x, edge_index, params):
    mask = jax.random.bernoulli(jax.random.key(42), 0.8, edge_index.shape)
    edge = edge_index * mask.astype(edge_index.dtype)
    npad = EPAD - edge_index.shape[1]
    pad = jnp.full((npad,), NUM_HE, jnp.int32)
    node = jnp.concatenate([edge[0], pad]).reshape(NW, CPW, CHUNK)
    he = jnp.concatenate([edge[1], pad]).reshape(NW, CPW, CHUNK)

    d_parts, b_parts = _sc_degrees(node, he)

    x_lo = x[:NUM_HE]
    x0 = _tc_matmul(x_lo, params['W_ln'], 5000, 1000)

    def row(v):
        return v[None, :]

    def hconv(h5000, wmat):
        C = wmat.shape[1]
        h_pad = jnp.pad(h5000, ((0, RPAD - NUM_HE), (0, 0)))
        tab = _tc_matmul(h_pad, wmat, RPAD, 1024)
        m_parts = _sc_pass(C)(tab, node, he)
        m2 = _tc_mid(m_parts, b_parts, C)
        return _sc_pass(C)(m2, he, node)

    oB = hconv(x_lo, params['W_c1'])
    h = _tc_stage1(oB, d_parts, x0, row(params['b_c1']),
                   params['W_fc1'], row(params['b_fc1']),
                   params['W_fc2'], row(params['b_fc2']),
                   row(params['g1']), row(params['beta1']))
    for i in range(3):
        oB = hconv(h, params['W_convs'][i])
        h = _tc_stageL(oB, d_parts, x0, row(params['b_convs'][i]),
                       row(params['gs'][i]), row(params['betas'][i]),
                       params['W_fc1s'][i], row(params['b_fc1s'][i]),
                       params['W_fc2s'][i], row(params['b_fc2s'][i]))
    oB = hconv(h, params['W_c3'])
    return _tc_final(oB, d_parts, row(params['b_c3']))
